# baseline (speedup 1.0000x reference)
import functools

import jax
import jax.numpy as jnp
from jax import lax
from jax.experimental import pallas as pl
from jax.experimental.pallas import tpu as pltpu
from jax.experimental.pallas import tpu_sc as plsc

_NC = 2
_NS = 16
_NW = _NC * _NS

_B, _C, _H, _W, _D = 4, 256, 32, 32, 32
_CH = 86
_HWD = _H * _W * _D
_CPW = _C // _NW


_TAB = 50 * _CH


def _pos_embed_sc(tabs):
    mesh = plsc.VectorSubcoreMesh(
        core_axis_name="c", subcore_axis_name="s",
        num_cores=_NC, num_subcores=_NS)

    @functools.partial(
        pl.kernel,
        out_type=jax.ShapeDtypeStruct((_B * _C * _HWD,), jnp.float32),
        mesh=mesh,
        scratch_types=[
            pltpu.VMEM((3 * _TAB,), jnp.float32),
            pltpu.VMEM((2, _HWD), jnp.float32),
            pltpu.SemaphoreType.DMA,
            pltpu.SemaphoreType.DMA,
        ],
        compiler_params=pltpu.CompilerParams(needs_layout_passes=False),
    )
    def body(tabs_hbm, out_hbm, tabs_v, buf_v, sem0, sem1):
        wid = lax.axis_index("s") * _NC + lax.axis_index("c")

        pltpu.sync_copy(tabs_hbm, tabs_v)

        lanes = lax.iota(jnp.int32, 16)
        sems = (sem0, sem1)
        pending = [[], []]

        for cc in range(_CPW):
            c = wid + _NW * cc
            slot = cc % 2
            for d in pending[slot]:
                d.wait()
            pending[slot] = []

            is_a = c < _CH
            is_b = jnp.logical_and(c >= _CH, c < 2 * _CH)
            g = jnp.where(is_a, 0, jnp.where(is_b, 1, 2)).astype(jnp.int32)
            coff = (c - g * _CH).astype(jnp.int32)
            base = g * _TAB + coff
            bvec = jnp.full((16,), base, jnp.int32)

            def build_a(bvec=bvec, slot=slot):
                def slab(i, _):
                    val = plsc.load_gather(tabs_v, [bvec + i * _CH])

                    @plsc.parallel_loop(0, 64, unroll=8)
                    def st(q):
                        buf_v[slot, pl.ds(i * 1024 + q * 16, 16)] = val
                    return 0
                lax.fori_loop(0, 32, slab, 0)

            def build_b(bvec=bvec, slot=slot):
                def jloop(j, _):
                    val = plsc.load_gather(tabs_v, [bvec + j * _CH])
                    buf_v[slot, pl.ds(j * 32, 16)] = val
                    buf_v[slot, pl.ds(j * 32 + 16, 16)] = val
                    return 0
                lax.fori_loop(0, 32, jloop, 0)
                n = 1024
                while n < _HWD:
                    @plsc.parallel_loop(0, n // 16, unroll=8)
                    def cp(q, n=n, slot=slot):
                        buf_v[slot, pl.ds(n + q * 16, 16)] = (
                            buf_v[slot, pl.ds(q * 16, 16)])
                    n *= 2

            def build_c(bvec=bvec, slot=slot):
                v0 = plsc.load_gather(tabs_v, [bvec + lanes * _CH])
                v1 = plsc.load_gather(tabs_v, [bvec + (lanes + 16) * _CH])

                @plsc.parallel_loop(0, 1024, unroll=8)
                def st(m):
                    buf_v[slot, pl.ds(m * 32, 16)] = v0
                    buf_v[slot, pl.ds(m * 32 + 16, 16)] = v1

            pl.when(is_a)(build_a)
            pl.when(is_b)(build_b)
            pl.when(jnp.logical_and(jnp.logical_not(is_a),
                                    jnp.logical_not(is_b)))(build_c)

            for b in range(_B):
                dst = out_hbm.at[pl.ds((b * _C + c) * _HWD, _HWD)]
                pending[slot].append(
                    pltpu.async_copy(buf_v.at[slot], dst, sems[slot]))

        for slot in range(2):
            for d in pending[slot]:
                d.wait()

    return body(tabs)


def kernel(tensor, row_embed, col_embed, dep_embed):
    b, c, h, w, d = tensor.shape
    tabs = jnp.concatenate(
        [col_embed.reshape(-1), row_embed.reshape(-1), dep_embed.reshape(-1)])
    out_flat = _pos_embed_sc(tabs)
    return out_flat.reshape(b, c, h * w * d)
